# Initial kernel scaffold; baseline (speedup 1.0000x reference)
import jax
import jax.numpy as jnp
from jax.experimental import pallas as pl


def kernel(x, edge_index, W1, a_src1, a_dst1, b1, W2, a_src2, a_dst2, b2, Wout, bout):
    raise NotImplementedError("write your pallas kernel here")


# speedup vs baseline: 17.6852x; 17.6852x over previous
import functools

import jax
import jax.numpy as jnp
from jax import lax
from jax.experimental import pallas as pl
from jax.experimental.pallas import tpu as pltpu
from jax.experimental.pallas import tpu_sc as plsc

N = 10000
E0 = 320000
E = E0 + N
D_IN = 128
DH = 64
SUB = 10

NC = 2
NS = 16
NW = NC * NS
R = 81
ROWS = NW * R
E_PAD = ROWS * 128
NPAD = 10240
NSL = NPAD // NS

_f32 = jnp.float32
_i32 = jnp.int32


def _tc_head_body(x_ref, w_ref, asrc_ref, adst_ref, h_ref, ss_ref, sd_ref, c_ref):
    h = jnp.dot(x_ref[...], w_ref[...], preferred_element_type=_f32)
    h_ref[...] = h
    ss = jnp.sum(h * asrc_ref[...][None, :], axis=1)
    sd = jnp.sum(h * adst_ref[...][None, :], axis=1)
    ss_ref[...] = ss
    sd_ref[...] = sd
    craw = jnp.max(ss) + jnp.max(sd)
    c = jnp.where(craw > 0.0, craw, 0.2 * craw)
    c_ref[...] = jnp.full((16,), c, _f32)


def _tc_head(x, w, asrc, adst):
    return pl.pallas_call(
        _tc_head_body,
        out_shape=[
            jax.ShapeDtypeStruct((N, DH), _f32),
            jax.ShapeDtypeStruct((N,), _f32),
            jax.ShapeDtypeStruct((N,), _f32),
            jax.ShapeDtypeStruct((16,), _f32),
        ],
    )(x, w, asrc, adst)


def _tc_mid_body(p0_ref, p1_ref, b_ref, w_ref, asrc_ref, adst_ref,
                 h_ref, ss_ref, sd_ref, c_ref):
    hin = jnp.maximum(p0_ref[...] + p1_ref[...] + b_ref[...][None, :], 0.0)
    h = jnp.dot(hin, w_ref[...], preferred_element_type=_f32)
    h_ref[...] = h
    ss = jnp.sum(h * asrc_ref[...][None, :], axis=1)
    sd = jnp.sum(h * adst_ref[...][None, :], axis=1)
    ss_ref[...] = ss
    sd_ref[...] = sd
    craw = jnp.max(ss) + jnp.max(sd)
    c = jnp.where(craw > 0.0, craw, 0.2 * craw)
    c_ref[...] = jnp.full((16,), c, _f32)


def _tc_mid(p0, p1, b, w, asrc, adst):
    return pl.pallas_call(
        _tc_mid_body,
        out_shape=[
            jax.ShapeDtypeStruct((N, DH), _f32),
            jax.ShapeDtypeStruct((N,), _f32),
            jax.ShapeDtypeStruct((N,), _f32),
            jax.ShapeDtypeStruct((16,), _f32),
        ],
    )(p0, p1, b, w, asrc, adst)


def _tc_rowsum_body(p0_ref, p1_ref, b_ref, wrep_ref, rs_ref):
    h = jnp.maximum(p0_ref[...] + p1_ref[...] + b_ref[...][None, :], 0.0)
    rs_ref[...] = jnp.sum(h * wrep_ref[...], axis=1)


def _tc_rowsum(p0, p1, b, wrep):
    return pl.pallas_call(
        _tc_rowsum_body,
        out_shape=jax.ShapeDtypeStruct((N,), _f32),
    )(p0, p1, b, wrep)


def _tc_fold_body(p_ref, bout_ref, o_ref):
    o_ref[...] = jnp.sum(p_ref[...], axis=1, keepdims=True) + bout_ref[...][None, :]


def _tc_fold(p, bout):
    return pl.pallas_call(
        _tc_fold_body,
        out_shape=jax.ShapeDtypeStruct((N // SUB, 1), _f32),
    )(p, bout)


_mesh = plsc.VectorSubcoreMesh(core_axis_name="c", subcore_axis_name="s")
_sc_params = pltpu.CompilerParams(use_tc_tiling_on_sc=False,
                                  needs_layout_passes=False)


@functools.partial(
    pl.kernel,
    out_type=[
        jax.ShapeDtypeStruct((ROWS, 128), _f32),
        jax.ShapeDtypeStruct((NPAD,), _f32),
        jax.ShapeDtypeStruct((NPAD,), _f32),
    ],
    mesh=_mesh,
    compiler_params=_sc_params,
    scratch_types=[
        pltpu.VMEM((R, 128), _i32),
        pltpu.VMEM((R, 128), _i32),
        pltpu.VMEM((R, 128), _f32),
        pltpu.VMEM((R, 128), _f32),
        pltpu.VMEM((16,), _f32),
        pltpu.VMEM((NSL,), _f32),
        pltpu.VMEM_SHARED((NPAD,), _f32),
        pltpu.SemaphoreType.DMA,
        pltpu.SemaphoreType.DMA,
    ],
)
def _sc_edge_softmax(src_hbm, dst_hbm, ss_hbm, sd_hbm, cv_hbm,
                     p_hbm, d0_hbm, d1_hbm,
                     vsrc, vdst, gs, gd, cbuf, zbuf, dsh, sem1, sem2):
    c = lax.axis_index("c")
    s = lax.axis_index("s")
    wid = c * NS + s
    base = wid * R

    pltpu.sync_copy(src_hbm.at[pl.ds(base, R)], vsrc)
    pltpu.sync_copy(dst_hbm.at[pl.ds(base, R)], vdst)
    pltpu.sync_copy(cv_hbm, cbuf)

    for t in range(NSL // 16):
        zbuf[pl.ds(t * 16, 16)] = jnp.zeros((16,), _f32)
    pltpu.sync_copy(zbuf, dsh.at[pl.ds(s * NSL, NSL)])

    def fire(j, carry):
        pltpu.async_copy(ss_hbm.at[vsrc.at[j]], gs.at[j], sem1)
        pltpu.async_copy(sd_hbm.at[vdst.at[j]], gd.at[j], sem1)
        return carry

    lax.fori_loop(0, R, fire, 0)
    pltpu.make_async_copy(p_hbm.at[pl.ds(base, R)], gs, sem1).wait()
    pltpu.make_async_copy(p_hbm.at[pl.ds(base, R)], gd, sem1).wait()
    cv = cbuf[...]

    plsc.subcore_barrier()

    def body(j, carry):
        for k in range(8):
            sl = pl.ds(k * 16, 16)
            e = gs[j, sl] + gd[j, sl]
            e = jnp.where(e > 0.0, e, 0.2 * e)
            p = jnp.exp(e - cv)
            eid = (base + j) * 128 + k * 16 + lax.iota(_i32, 16)
            p = jnp.where(eid < E, p, 0.0)
            gs[j, sl] = p
        return carry

    lax.fori_loop(0, R, body, 0)

    pltpu.sync_copy(gs, p_hbm.at[pl.ds(base, R)])

    def fire_add(j, carry):
        pltpu.async_copy(gs.at[j], dsh.at[vdst.at[j]], sem2, add=True)
        return carry

    lax.fori_loop(0, R, fire_add, 0)
    pltpu.make_async_copy(p_hbm.at[pl.ds(base, R)], gs, sem2).wait()
    plsc.subcore_barrier()

    sl6 = pl.ds(s * NSL, NSL)

    @pl.when(c == 0)
    def _():
        pltpu.sync_copy(dsh.at[sl6], d0_hbm.at[sl6])

    @pl.when(c == 1)
    def _():
        pltpu.sync_copy(dsh.at[sl6], d1_hbm.at[sl6])


@functools.partial(
    pl.kernel,
    out_type=[
        jax.ShapeDtypeStruct((NPAD, DH), _f32),
        jax.ShapeDtypeStruct((NPAD, DH), _f32),
    ],
    mesh=_mesh,
    compiler_params=_sc_params,
    scratch_types=[
        pltpu.VMEM((R, 128), _i32),
        pltpu.VMEM((R, 128), _i32),
        pltpu.VMEM((R, 128), _f32),
        pltpu.VMEM((R, 128), _f32),
        pltpu.VMEM((R, 128), _f32),
        pltpu.VMEM((128, DH), _f32),
        pltpu.VMEM((64, DH), _f32),
        pltpu.VMEM_SHARED((NPAD, DH), _f32),
        pltpu.SemaphoreType.DMA,
        pltpu.SemaphoreType.DMA,
    ],
)
def _sc_aggregate(src_hbm, dst_hbm, p_hbm, d0_hbm, d1_hbm, h_hbm,
                  o0_hbm, o1_hbm,
                  vsrc, vdst, vp, g0, g1, rows, zbuf, osh, sem1, sem2):
    c = lax.axis_index("c")
    s = lax.axis_index("s")
    wid = c * NS + s
    base = wid * R

    pltpu.sync_copy(src_hbm.at[pl.ds(base, R)], vsrc)
    pltpu.sync_copy(dst_hbm.at[pl.ds(base, R)], vdst)
    pltpu.sync_copy(p_hbm.at[pl.ds(base, R)], vp)

    for r in range(64):
        for t in range(DH // 16):
            zbuf[r, pl.ds(t * 16, 16)] = jnp.zeros((16,), _f32)
    row0 = s * NSL
    for t in range(NSL // 64):
        pltpu.sync_copy(zbuf, osh.at[pl.ds(row0 + t * 64, 64)])

    def fire(j, carry):
        pltpu.async_copy(d0_hbm.at[vdst.at[j]], g0.at[j], sem2)
        pltpu.async_copy(d1_hbm.at[vdst.at[j]], g1.at[j], sem2)
        return carry

    lax.fori_loop(0, R, fire, 0)
    pltpu.make_async_copy(p_hbm.at[pl.ds(base, R)], g0, sem2).wait()
    pltpu.make_async_copy(p_hbm.at[pl.ds(base, R)], g1, sem2).wait()

    def alpha_body(j, carry):
        for k in range(8):
            sl = pl.ds(k * 16, 16)
            vp[j, sl] = vp[j, sl] / (g0[j, sl] + g1[j, sl] + 1e-16)
        return carry

    lax.fori_loop(0, R, alpha_body, 0)

    plsc.subcore_barrier()

    def body(j, carry):
        cp = pltpu.async_copy(h_hbm.at[vsrc.at[j]], rows, sem1)
        cp.wait()
        jv = jnp.full((16,), j, _i32)
        for i in range(128):
            a = plsc.load_gather(vp, [jv, jnp.full((16,), i, _i32)])
            for t in range(DH // 16):
                sl = pl.ds(t * 16, 16)
                rows[i, sl] = rows[i, sl] * a
        pltpu.sync_copy(rows, osh.at[vdst.at[j]], add=True)
        return carry

    lax.fori_loop(0, R, body, 0)

    plsc.subcore_barrier()
    slr = pl.ds(row0, NSL)

    @pl.when(c == 0)
    def _():
        pltpu.sync_copy(osh.at[slr], o0_hbm.at[slr])

    @pl.when(c == 1)
    def _():
        pltpu.sync_copy(osh.at[slr], o1_hbm.at[slr])


def kernel(x, edge_index, W1, a_src1, a_dst1, b1, W2, a_src2, a_dst2, b2,
           Wout, bout):
    loop = jnp.arange(N, dtype=_i32)
    padi = jnp.zeros((E_PAD - E,), _i32)
    src = jnp.concatenate([edge_index[0].astype(_i32), loop, padi]).reshape(ROWS, 128)
    dst = jnp.concatenate([edge_index[1].astype(_i32), loop, padi]).reshape(ROWS, 128)

    h1, ss1, sd1, c1 = _tc_head(x, W1, a_src1, a_dst1)
    p1, d0, d1 = _sc_edge_softmax(src, dst, ss1, sd1, c1)
    o0, o1 = _sc_aggregate(src, dst, p1, d0, d1, h1)

    h2, ss2, sd2, c2 = _tc_mid(o0[:N], o1[:N], b1, W2, a_src2, a_dst2)
    p2, e0, e1 = _sc_edge_softmax(src, dst, ss2, sd2, c2)
    q0, q1 = _sc_aggregate(src, dst, p2, e0, e1, h2)

    wrep = jnp.tile(jnp.reshape(Wout[:, 0], (SUB, DH)), (N // SUB, 1))
    rs = _tc_rowsum(q0[:N], q1[:N], b2, wrep)
    out = _tc_fold(jnp.reshape(rs, (N // SUB, SUB)), bout)
    return out


# speedup vs baseline: 19.8706x; 1.1236x over previous
import functools

import jax
import jax.numpy as jnp
from jax import lax
from jax.experimental import pallas as pl
from jax.experimental.pallas import tpu as pltpu
from jax.experimental.pallas import tpu_sc as plsc

N = 10000
E0 = 320000
E = E0 + N
D_IN = 128
DH = 64
SUB = 10

NC = 2
NS = 16
NW = NC * NS
R = 81
ROWS = NW * R
E_PAD = ROWS * 128
NPAD = 10240
NSL = NPAD // NS

_f32 = jnp.float32
_i32 = jnp.int32


def _tc_head_body(x_ref, w_ref, asrc_ref, adst_ref, h_ref, ss_ref, sd_ref, c_ref):
    h = jnp.dot(x_ref[...], w_ref[...], preferred_element_type=_f32)
    h_ref[...] = h
    ss = jnp.sum(h * asrc_ref[...][None, :], axis=1)
    sd = jnp.sum(h * adst_ref[...][None, :], axis=1)
    ss_ref[...] = ss
    sd_ref[...] = sd
    craw = jnp.max(ss) + jnp.max(sd)
    c = jnp.where(craw > 0.0, craw, 0.2 * craw)
    c_ref[...] = jnp.full((16,), c, _f32)


def _tc_head(x, w, asrc, adst):
    return pl.pallas_call(
        _tc_head_body,
        out_shape=[
            jax.ShapeDtypeStruct((N, DH), _f32),
            jax.ShapeDtypeStruct((N,), _f32),
            jax.ShapeDtypeStruct((N,), _f32),
            jax.ShapeDtypeStruct((16,), _f32),
        ],
    )(x, w, asrc, adst)


def _tc_mid_body(p0_ref, p1_ref, b_ref, w_ref, asrc_ref, adst_ref,
                 h_ref, ss_ref, sd_ref, c_ref):
    hin = jnp.maximum(p0_ref[...] + p1_ref[...] + b_ref[...][None, :], 0.0)
    h = jnp.dot(hin, w_ref[...], preferred_element_type=_f32)
    h_ref[...] = h
    ss = jnp.sum(h * asrc_ref[...][None, :], axis=1)
    sd = jnp.sum(h * adst_ref[...][None, :], axis=1)
    ss_ref[...] = ss
    sd_ref[...] = sd
    craw = jnp.max(ss) + jnp.max(sd)
    c = jnp.where(craw > 0.0, craw, 0.2 * craw)
    c_ref[...] = jnp.full((16,), c, _f32)


def _tc_mid(p0, p1, b, w, asrc, adst):
    return pl.pallas_call(
        _tc_mid_body,
        out_shape=[
            jax.ShapeDtypeStruct((N, DH), _f32),
            jax.ShapeDtypeStruct((N,), _f32),
            jax.ShapeDtypeStruct((N,), _f32),
            jax.ShapeDtypeStruct((16,), _f32),
        ],
    )(p0, p1, b, w, asrc, adst)


def _tc_rowsum_body(p0_ref, p1_ref, b_ref, wrep_ref, rs_ref):
    h = jnp.maximum(p0_ref[...] + p1_ref[...] + b_ref[...][None, :], 0.0)
    rs_ref[...] = jnp.sum(h * wrep_ref[...], axis=1)


def _tc_rowsum(p0, p1, b, wrep):
    return pl.pallas_call(
        _tc_rowsum_body,
        out_shape=jax.ShapeDtypeStruct((N,), _f32),
    )(p0, p1, b, wrep)


def _tc_fold_body(p_ref, bout_ref, o_ref):
    o_ref[...] = jnp.sum(p_ref[...], axis=1, keepdims=True) + bout_ref[...][None, :]


def _tc_fold(p, bout):
    return pl.pallas_call(
        _tc_fold_body,
        out_shape=jax.ShapeDtypeStruct((N // SUB, 1), _f32),
    )(p, bout)


_mesh = plsc.VectorSubcoreMesh(core_axis_name="c", subcore_axis_name="s")
_sc_params = pltpu.CompilerParams(use_tc_tiling_on_sc=False,
                                  needs_layout_passes=False)


@functools.partial(
    pl.kernel,
    out_type=[
        jax.ShapeDtypeStruct((ROWS, 128), _f32),
        jax.ShapeDtypeStruct((NPAD,), _f32),
        jax.ShapeDtypeStruct((NPAD,), _f32),
    ],
    mesh=_mesh,
    compiler_params=_sc_params,
    scratch_types=[
        pltpu.VMEM((R, 128), _i32),
        pltpu.VMEM((R, 128), _i32),
        pltpu.VMEM((R, 128), _f32),
        pltpu.VMEM((R, 128), _f32),
        pltpu.VMEM((16,), _f32),
        pltpu.VMEM((NSL,), _f32),
        pltpu.VMEM_SHARED((NPAD,), _f32),
        pltpu.SemaphoreType.DMA,
        pltpu.SemaphoreType.DMA,
    ],
)
def _sc_edge_softmax(src_hbm, dst_hbm, ss_hbm, sd_hbm, cv_hbm,
                     p_hbm, d0_hbm, d1_hbm,
                     vsrc, vdst, gs, gd, cbuf, zbuf, dsh, sem1, sem2):
    c = lax.axis_index("c")
    s = lax.axis_index("s")
    wid = c * NS + s
    base = wid * R

    pltpu.sync_copy(src_hbm.at[pl.ds(base, R)], vsrc)
    pltpu.sync_copy(dst_hbm.at[pl.ds(base, R)], vdst)
    pltpu.sync_copy(cv_hbm, cbuf)

    for t in range(NSL // 16):
        zbuf[pl.ds(t * 16, 16)] = jnp.zeros((16,), _f32)
    pltpu.sync_copy(zbuf, dsh.at[pl.ds(s * NSL, NSL)])

    def fire(j, carry):
        pltpu.async_copy(ss_hbm.at[vsrc.at[j]], gs.at[j], sem1)
        pltpu.async_copy(sd_hbm.at[vdst.at[j]], gd.at[j], sem1)
        return carry

    lax.fori_loop(0, R, fire, 0)
    pltpu.make_async_copy(p_hbm.at[pl.ds(base, R)], gs, sem1).wait()
    pltpu.make_async_copy(p_hbm.at[pl.ds(base, R)], gd, sem1).wait()
    cv = cbuf[...]

    plsc.subcore_barrier()

    def body(j, carry):
        for k in range(8):
            sl = pl.ds(k * 16, 16)
            e = gs[j, sl] + gd[j, sl]
            e = jnp.where(e > 0.0, e, 0.2 * e)
            gs[j, sl] = jnp.exp(e - cv)
        return carry

    lax.fori_loop(0, R, body, 0)

    @pl.when(wid == NW - 1)
    def _():
        zero = jnp.zeros((16,), _f32)
        pad0 = E - (NW - 1) * R * 128
        jpad, kpad = pad0 // 128, pad0 % 128
        for k in range(kpad // 16, 8):
            gs[jpad, pl.ds(k * 16, 16)] = zero

        def zrow(j, carry):
            for k in range(8):
                gs[j, pl.ds(k * 16, 16)] = zero
            return carry

        lax.fori_loop(jpad + 1, R, zrow, 0)

    pltpu.sync_copy(gs, p_hbm.at[pl.ds(base, R)])

    def fire_add(j, carry):
        pltpu.async_copy(gs.at[j], dsh.at[vdst.at[j]], sem2, add=True)
        return carry

    lax.fori_loop(0, R, fire_add, 0)
    pltpu.make_async_copy(p_hbm.at[pl.ds(base, R)], gs, sem2).wait()
    plsc.subcore_barrier()

    sl6 = pl.ds(s * NSL, NSL)

    @pl.when(c == 0)
    def _():
        pltpu.sync_copy(dsh.at[sl6], d0_hbm.at[sl6])

    @pl.when(c == 1)
    def _():
        pltpu.sync_copy(dsh.at[sl6], d1_hbm.at[sl6])


@functools.partial(
    pl.kernel,
    out_type=[
        jax.ShapeDtypeStruct((NPAD, DH), _f32),
        jax.ShapeDtypeStruct((NPAD, DH), _f32),
    ],
    mesh=_mesh,
    compiler_params=_sc_params,
    scratch_types=[
        pltpu.VMEM((R, 128), _i32),
        pltpu.VMEM((R, 128), _i32),
        pltpu.VMEM((R, 128), _f32),
        pltpu.VMEM((R, 128), _f32),
        pltpu.VMEM((R, 128), _f32),
        pltpu.VMEM((128, DH), _f32),
        pltpu.VMEM((128, DH), _f32),
        pltpu.VMEM((128, DH), _f32),
        pltpu.VMEM((128, DH), _f32),
        pltpu.VMEM((64, DH), _f32),
        pltpu.VMEM_SHARED((NPAD, DH), _f32),
        pltpu.SemaphoreType.DMA,
        pltpu.SemaphoreType.DMA,
        pltpu.SemaphoreType.DMA,
        pltpu.SemaphoreType.DMA,
        pltpu.SemaphoreType.DMA,
        pltpu.SemaphoreType.DMA,
    ],
)
def _sc_aggregate(src_hbm, dst_hbm, p_hbm, d0_hbm, d1_hbm, h_hbm,
                  o0_hbm, o1_hbm,
                  vsrc, vdst, vp, g0, g1, rin0, rin1, rout0, rout1,
                  zbuf, osh, sem1, sem2, semg0, semg1, sems0, sems1):
    c = lax.axis_index("c")
    s = lax.axis_index("s")
    wid = c * NS + s
    base = wid * R

    pltpu.sync_copy(src_hbm.at[pl.ds(base, R)], vsrc)
    pltpu.sync_copy(dst_hbm.at[pl.ds(base, R)], vdst)
    pltpu.sync_copy(p_hbm.at[pl.ds(base, R)], vp)

    for r in range(64):
        for t in range(DH // 16):
            zbuf[r, pl.ds(t * 16, 16)] = jnp.zeros((16,), _f32)
    row0 = s * NSL
    for t in range(NSL // 64):
        pltpu.sync_copy(zbuf, osh.at[pl.ds(row0 + t * 64, 64)])

    def fire(j, carry):
        pltpu.async_copy(d0_hbm.at[vdst.at[j]], g0.at[j], sem2)
        pltpu.async_copy(d1_hbm.at[vdst.at[j]], g1.at[j], sem2)
        return carry

    lax.fori_loop(0, R, fire, 0)
    pltpu.make_async_copy(p_hbm.at[pl.ds(base, R)], g0, sem2).wait()
    pltpu.make_async_copy(p_hbm.at[pl.ds(base, R)], g1, sem2).wait()

    def alpha_body(j, carry):
        for k in range(8):
            sl = pl.ds(k * 16, 16)
            vp[j, sl] = vp[j, sl] / (g0[j, sl] + g1[j, sl] + 1e-16)
        return carry

    lax.fori_loop(0, R, alpha_body, 0)

    plsc.subcore_barrier()

    def scale(j, rin, rout):
        jv = jnp.full((16,), j, _i32)
        for i in range(128):
            a = plsc.load_gather(vp, [jv, jnp.full((16,), i, _i32)])
            for t in range(DH // 16):
                sl = pl.ds(t * 16, 16)
                rout[i, sl] = rin[i, sl] * a

    def fire_gather(j, rin, semg):
        pltpu.async_copy(h_hbm.at[vsrc.at[j]], rin, semg)

    def fire_scatter(j, rout, sems):
        pltpu.async_copy(rout, osh.at[vdst.at[j]], sems, add=True)

    def drain(buf, sem):
        pltpu.make_async_copy(h_hbm.at[pl.ds(0, 128)], buf, sem).wait()

    fire_gather(0, rin0, semg0)
    fire_gather(1, rin1, semg1)

    drain(rin0, semg0)
    scale(0, rin0, rout0)
    fire_scatter(0, rout0, sems0)
    fire_gather(2, rin0, semg0)
    drain(rin1, semg1)
    scale(1, rin1, rout1)
    fire_scatter(1, rout1, sems1)
    fire_gather(3, rin1, semg1)

    def body(g, carry):
        j0 = 2 * g
        drain(rin0, semg0)
        drain(rout0, sems0)
        scale(j0, rin0, rout0)
        fire_scatter(j0, rout0, sems0)
        fire_gather(j0 + 2, rin0, semg0)
        drain(rin1, semg1)
        drain(rout1, sems1)
        scale(j0 + 1, rin1, rout1)
        fire_scatter(j0 + 1, rout1, sems1)

        @pl.when(j0 + 3 < R)
        def _():
            fire_gather(j0 + 3, rin1, semg1)

        return carry

    lax.fori_loop(1, (R - 1) // 2, body, 0)

    drain(rin0, semg0)
    drain(rout0, sems0)
    scale(R - 1, rin0, rout0)
    fire_scatter(R - 1, rout0, sems0)
    drain(rout0, sems0)
    drain(rout1, sems1)

    plsc.subcore_barrier()
    slr = pl.ds(row0, NSL)

    @pl.when(c == 0)
    def _():
        pltpu.sync_copy(osh.at[slr], o0_hbm.at[slr])

    @pl.when(c == 1)
    def _():
        pltpu.sync_copy(osh.at[slr], o1_hbm.at[slr])


def kernel(x, edge_index, W1, a_src1, a_dst1, b1, W2, a_src2, a_dst2, b2,
           Wout, bout):
    loop = jnp.arange(N, dtype=_i32)
    padi = jnp.zeros((E_PAD - E,), _i32)
    src = jnp.concatenate([edge_index[0].astype(_i32), loop, padi]).reshape(ROWS, 128)
    dst = jnp.concatenate([edge_index[1].astype(_i32), loop, padi]).reshape(ROWS, 128)

    h1, ss1, sd1, c1 = _tc_head(x, W1, a_src1, a_dst1)
    p1, d0, d1 = _sc_edge_softmax(src, dst, ss1, sd1, c1)
    o0, o1 = _sc_aggregate(src, dst, p1, d0, d1, h1)

    h2, ss2, sd2, c2 = _tc_mid(o0[:N], o1[:N], b1, W2, a_src2, a_dst2)
    p2, e0, e1 = _sc_edge_softmax(src, dst, ss2, sd2, c2)
    q0, q1 = _sc_aggregate(src, dst, p2, e0, e1, h2)

    wrep = jnp.tile(jnp.reshape(Wout[:, 0], (SUB, DH)), (N // SUB, 1))
    rs = _tc_rowsum(q0[:N], q1[:N], b2, wrep)
    out = _tc_fold(jnp.reshape(rs, (N // SUB, SUB)), bout)
    return out


# speedup vs baseline: 37.4771x; 1.8861x over previous
import functools

import jax
import jax.numpy as jnp
from jax import lax
from jax.experimental import pallas as pl
from jax.experimental.pallas import tpu as pltpu
from jax.experimental.pallas import tpu_sc as plsc

N = 10000
E0 = 320000
E = E0 + N
D_IN = 128
DH = 64
SUB = 10

NC = 2
NS = 16
NW = NC * NS
R = 81
ROWS = NW * R
E_PAD = ROWS * 128
NPAD = 10240
NSL = NPAD // NS

_f32 = jnp.float32
_i32 = jnp.int32


def _tc_head_body(x_ref, w_ref, asrc_ref, adst_ref, h_ref, ss_ref, sd_ref, c_ref):
    h = jnp.dot(x_ref[...], w_ref[...], preferred_element_type=_f32)
    h_ref[...] = h
    ss = jnp.sum(h * asrc_ref[...][None, :], axis=1)
    sd = jnp.sum(h * adst_ref[...][None, :], axis=1)
    ss_ref[...] = ss
    sd_ref[...] = sd
    craw = jnp.max(ss) + jnp.max(sd)
    c = jnp.where(craw > 0.0, craw, 0.2 * craw)
    c_ref[...] = jnp.full((16,), c, _f32)


def _tc_head(x, w, asrc, adst):
    return pl.pallas_call(
        _tc_head_body,
        out_shape=[
            jax.ShapeDtypeStruct((N, DH), _f32),
            jax.ShapeDtypeStruct((N,), _f32),
            jax.ShapeDtypeStruct((N,), _f32),
            jax.ShapeDtypeStruct((16,), _f32),
        ],
    )(x, w, asrc, adst)


def _tc_mid_body(p0_ref, p1_ref, d0_ref, d1_ref, b_ref, w_ref, asrc_ref, adst_ref,
                 h_ref, ss_ref, sd_ref, c_ref):
    den = d0_ref[...] + d1_ref[...] + 1e-16
    hin = jnp.maximum((p0_ref[...] + p1_ref[...]) / den + b_ref[...][None, :], 0.0)
    h = jnp.dot(hin, w_ref[...], preferred_element_type=_f32)
    h_ref[...] = h
    ss = jnp.sum(h * asrc_ref[...][None, :], axis=1)
    sd = jnp.sum(h * adst_ref[...][None, :], axis=1)
    ss_ref[...] = ss
    sd_ref[...] = sd
    craw = jnp.max(ss) + jnp.max(sd)
    c = jnp.where(craw > 0.0, craw, 0.2 * craw)
    c_ref[...] = jnp.full((16,), c, _f32)


def _tc_mid(p0, p1, d0, d1, b, w, asrc, adst):
    return pl.pallas_call(
        _tc_mid_body,
        out_shape=[
            jax.ShapeDtypeStruct((N, DH), _f32),
            jax.ShapeDtypeStruct((N,), _f32),
            jax.ShapeDtypeStruct((N,), _f32),
            jax.ShapeDtypeStruct((16,), _f32),
        ],
    )(p0, p1, d0, d1, b, w, asrc, adst)


def _tc_rowsum_body(p0_ref, p1_ref, d0_ref, d1_ref, b_ref, wrep_ref, rs_ref):
    den = d0_ref[...] + d1_ref[...] + 1e-16
    h = jnp.maximum((p0_ref[...] + p1_ref[...]) / den + b_ref[...][None, :], 0.0)
    rs_ref[...] = jnp.sum(h * wrep_ref[...], axis=1)


def _tc_rowsum(p0, p1, d0, d1, b, wrep):
    return pl.pallas_call(
        _tc_rowsum_body,
        out_shape=jax.ShapeDtypeStruct((N,), _f32),
    )(p0, p1, d0, d1, b, wrep)


def _tc_fold_body(p_ref, bout_ref, o_ref):
    o_ref[...] = jnp.sum(p_ref[...], axis=1, keepdims=True) + bout_ref[...][None, :]


def _tc_fold(p, bout):
    return pl.pallas_call(
        _tc_fold_body,
        out_shape=jax.ShapeDtypeStruct((N // SUB, 1), _f32),
    )(p, bout)


_mesh = plsc.VectorSubcoreMesh(core_axis_name="c", subcore_axis_name="s")
_sc_params = pltpu.CompilerParams(use_tc_tiling_on_sc=False,
                                  needs_layout_passes=False)


@functools.partial(
    pl.kernel,
    out_type=[
        jax.ShapeDtypeStruct((ROWS, 128), _f32),
        jax.ShapeDtypeStruct((NPAD,), _f32),
        jax.ShapeDtypeStruct((NPAD,), _f32),
    ],
    mesh=_mesh,
    compiler_params=_sc_params,
    scratch_types=[
        pltpu.VMEM((R, 128), _i32),
        pltpu.VMEM((R, 128), _i32),
        pltpu.VMEM((R, 128), _f32),
        pltpu.VMEM((N,), _f32),
        pltpu.VMEM((N,), _f32),
        pltpu.VMEM((16,), _f32),
        pltpu.VMEM((NSL,), _f32),
        pltpu.VMEM_SHARED((NPAD,), _f32),
        pltpu.SemaphoreType.DMA,
        pltpu.SemaphoreType.DMA,
    ],
)
def _sc_edge_softmax(src_hbm, dst_hbm, ss_hbm, sd_hbm, cv_hbm,
                     p_hbm, d0_hbm, d1_hbm,
                     vsrc, vdst, gs, ssv, sdv, cbuf, zbuf, dsh, sem1, sem2):
    c = lax.axis_index("c")
    s = lax.axis_index("s")
    wid = c * NS + s
    base = wid * R

    pltpu.async_copy(src_hbm.at[pl.ds(base, R)], vsrc, sem1)
    pltpu.async_copy(dst_hbm.at[pl.ds(base, R)], vdst, sem1)
    pltpu.async_copy(ss_hbm, ssv, sem1)
    pltpu.async_copy(sd_hbm, sdv, sem1)
    pltpu.sync_copy(cv_hbm, cbuf)

    for t in range(NSL // 16):
        zbuf[pl.ds(t * 16, 16)] = jnp.zeros((16,), _f32)
    pltpu.sync_copy(zbuf, dsh.at[pl.ds(s * NSL, NSL)])

    pltpu.make_async_copy(src_hbm.at[pl.ds(base, R)], vsrc, sem1).wait()
    pltpu.make_async_copy(src_hbm.at[pl.ds(base, R)], vdst, sem1).wait()
    pltpu.make_async_copy(ss_hbm, ssv, sem1).wait()
    pltpu.make_async_copy(sd_hbm, sdv, sem1).wait()
    cv = cbuf[...]

    plsc.subcore_barrier()

    def body(j, carry):
        for k in range(8):
            sl = pl.ds(k * 16, 16)
            e = (plsc.load_gather(ssv, [vsrc[j, sl]])
                 + plsc.load_gather(sdv, [vdst[j, sl]]))
            e = jnp.where(e > 0.0, e, 0.2 * e)
            gs[j, sl] = jnp.exp(e - cv)
        return carry

    lax.fori_loop(0, R, body, 0)

    @pl.when(wid == NW - 1)
    def _():
        zero = jnp.zeros((16,), _f32)
        pad0 = E - (NW - 1) * R * 128
        jpad, kpad = pad0 // 128, pad0 % 128
        for k in range(kpad // 16, 8):
            gs[jpad, pl.ds(k * 16, 16)] = zero

        def zrow(j, carry):
            for k in range(8):
                gs[j, pl.ds(k * 16, 16)] = zero
            return carry

        lax.fori_loop(jpad + 1, R, zrow, 0)

    pltpu.sync_copy(gs, p_hbm.at[pl.ds(base, R)])

    def fire_add(j, carry):
        pltpu.async_copy(gs.at[j], dsh.at[vdst.at[j]], sem2, add=True)
        return carry

    lax.fori_loop(0, R, fire_add, 0)
    pltpu.make_async_copy(p_hbm.at[pl.ds(base, R)], gs, sem2).wait()
    plsc.subcore_barrier()

    sl6 = pl.ds(s * NSL, NSL)

    @pl.when(c == 0)
    def _():
        pltpu.sync_copy(dsh.at[sl6], d0_hbm.at[sl6])

    @pl.when(c == 1)
    def _():
        pltpu.sync_copy(dsh.at[sl6], d1_hbm.at[sl6])


@functools.partial(
    pl.kernel,
    out_type=[
        jax.ShapeDtypeStruct((NPAD, DH), _f32),
        jax.ShapeDtypeStruct((NPAD, DH), _f32),
    ],
    mesh=_mesh,
    compiler_params=_sc_params,
    scratch_types=[
        pltpu.VMEM((R, 128), _i32),
        pltpu.VMEM((R, 128), _i32),
        pltpu.VMEM((R, 128), _f32),
        pltpu.VMEM((128, DH), _f32),
        pltpu.VMEM((128, DH), _f32),
        pltpu.VMEM((128, DH), _f32),
        pltpu.VMEM((128, DH), _f32),
        pltpu.VMEM((64, DH), _f32),
        pltpu.VMEM_SHARED((NPAD, DH), _f32),
        pltpu.SemaphoreType.DMA,
        pltpu.SemaphoreType.DMA,
        pltpu.SemaphoreType.DMA,
        pltpu.SemaphoreType.DMA,
    ],
)
def _sc_aggregate(src_hbm, dst_hbm, p_hbm, h_hbm,
                  o0_hbm, o1_hbm,
                  vsrc, vdst, vp, rin0, rin1, rout0, rout1,
                  zbuf, osh, semg0, semg1, sems0, sems1):
    c = lax.axis_index("c")
    s = lax.axis_index("s")
    wid = c * NS + s
    base = wid * R

    pltpu.sync_copy(src_hbm.at[pl.ds(base, R)], vsrc)
    pltpu.sync_copy(dst_hbm.at[pl.ds(base, R)], vdst)
    pltpu.sync_copy(p_hbm.at[pl.ds(base, R)], vp)

    for r in range(64):
        for t in range(DH // 16):
            zbuf[r, pl.ds(t * 16, 16)] = jnp.zeros((16,), _f32)
    row0 = s * NSL
    for t in range(NSL // 64):
        pltpu.sync_copy(zbuf, osh.at[pl.ds(row0 + t * 64, 64)])

    plsc.subcore_barrier()

    def scale(j, rin, rout):
        for g in range(8):
            av = vp[j, pl.ds(g * 16, 16)]
            for u in range(16):
                i = g * 16 + u
                a = av[u]
                for t in range(DH // 16):
                    sl = pl.ds(t * 16, 16)
                    rout[i, sl] = rin[i, sl] * a

    def fire_gather(j, rin, semg):
        pltpu.async_copy(h_hbm.at[vsrc.at[j]], rin, semg)

    def fire_scatter(j, rout, sems):
        pltpu.async_copy(rout, osh.at[vdst.at[j]], sems, add=True)

    def drain(buf, sem):
        pltpu.make_async_copy(h_hbm.at[pl.ds(0, 128)], buf, sem).wait()

    fire_gather(0, rin0, semg0)
    fire_gather(1, rin1, semg1)

    drain(rin0, semg0)
    scale(0, rin0, rout0)
    fire_scatter(0, rout0, sems0)
    fire_gather(2, rin0, semg0)
    drain(rin1, semg1)
    scale(1, rin1, rout1)
    fire_scatter(1, rout1, sems1)
    fire_gather(3, rin1, semg1)

    def body(g, carry):
        j0 = 2 * g
        drain(rin0, semg0)
        drain(rout0, sems0)
        scale(j0, rin0, rout0)
        fire_scatter(j0, rout0, sems0)
        fire_gather(j0 + 2, rin0, semg0)
        drain(rin1, semg1)
        drain(rout1, sems1)
        scale(j0 + 1, rin1, rout1)
        fire_scatter(j0 + 1, rout1, sems1)

        @pl.when(j0 + 3 < R)
        def _():
            fire_gather(j0 + 3, rin1, semg1)

        return carry

    lax.fori_loop(1, (R - 1) // 2, body, 0)

    drain(rin0, semg0)
    drain(rout0, sems0)
    scale(R - 1, rin0, rout0)
    fire_scatter(R - 1, rout0, sems0)
    drain(rout0, sems0)
    drain(rout1, sems1)

    plsc.subcore_barrier()
    slr = pl.ds(row0, NSL)

    @pl.when(c == 0)
    def _():
        pltpu.sync_copy(osh.at[slr], o0_hbm.at[slr])

    @pl.when(c == 1)
    def _():
        pltpu.sync_copy(osh.at[slr], o1_hbm.at[slr])


def kernel(x, edge_index, W1, a_src1, a_dst1, b1, W2, a_src2, a_dst2, b2,
           Wout, bout):
    loop = jnp.arange(N, dtype=_i32)
    padi = jnp.zeros((E_PAD - E,), _i32)
    src = jnp.concatenate([edge_index[0].astype(_i32), loop, padi]).reshape(ROWS, 128)
    dst = jnp.concatenate([edge_index[1].astype(_i32), loop, padi]).reshape(ROWS, 128)

    h1, ss1, sd1, c1 = _tc_head(x, W1, a_src1, a_dst1)
    p1, d0, d1 = _sc_edge_softmax(src, dst, ss1, sd1, c1)
    o0, o1 = _sc_aggregate(src, dst, p1, h1)

    d0r = jnp.reshape(d0[:N], (N, 1))
    d1r = jnp.reshape(d1[:N], (N, 1))
    h2, ss2, sd2, c2 = _tc_mid(o0[:N], o1[:N], d0r, d1r, b1, W2, a_src2, a_dst2)
    p2, e0, e1 = _sc_edge_softmax(src, dst, ss2, sd2, c2)
    q0, q1 = _sc_aggregate(src, dst, p2, h2)

    e0r = jnp.reshape(e0[:N], (N, 1))
    e1r = jnp.reshape(e1[:N], (N, 1))
    wrep = jnp.tile(jnp.reshape(Wout[:, 0], (SUB, DH)), (N // SUB, 1))
    rs = _tc_rowsum(q0[:N], q1[:N], e0r, e1r, b2, wrep)
    out = _tc_fold(jnp.reshape(rs, (N // SUB, SUB)), bout)
    return out
